# baseline (device time: 95441 ns/iter reference)
import jax
import jax.numpy as jnp
from jax import lax
from jax.experimental import pallas as pl
from jax.experimental.pallas import tpu as pltpu

N_DEV = 4
N_TOK = 2048
D_MODEL = 512
D_FF = 1024
E_LOCAL = 8
CHUNK = N_TOK // N_DEV
CAP = 640
KDIM = E_LOCAL * D_MODEL
BF = jnp.bfloat16
F32 = jnp.float32
INVALID = -1


def kernel(x, router_W, route_idx, expert_W, shared_W):
    owner = route_idx[:, 0] // E_LOCAL
    ind = (owner[:, None] == jnp.arange(N_DEV)[None, :]).astype(jnp.float32)
    rank = jnp.cumsum(ind, axis=0) - ind
    ar1 = (jnp.arange(N_TOK, dtype=jnp.float32) + 1.0)[None, :]
    slots = jnp.arange(CAP, dtype=jnp.float32)[None, :]
    rows = []
    for d in range(N_DEV):
        oh = (rank[:, d:d + 1] == slots) & (ind[:, d:d + 1] > 0)
        rows.append(
            jnp.dot(
                ar1,
                oh.astype(jnp.float32),
                precision=lax.Precision.HIGHEST,
            )
        )
    perm_dc = (jnp.concatenate(rows, axis=0) - 1.0).astype(jnp.int32)
    perm_cd = perm_dc.T

    def body(
        x_ref,
        router_ref,
        route_ref,
        ew_ref,
        sw_ref,
        pdc_ref,
        pcd_ref,
        out_ref,
        x_bf,
        b_bf,
        sw_bf,
        a_bf,
        ybuf,
        send_sems,
        recv_sems,
    ):
        my_pos = lax.axis_index("i")
        left = lax.rem(my_pos - 1 + N_DEV, N_DEV)
        right = lax.rem(my_pos + 1, N_DEV)

        def cmod(k):
            return lax.rem(my_pos + k + N_DEV, N_DEV)

        barrier_sem = pltpu.get_barrier_semaphore()
        for nbr in (left, right):
            pl.semaphore_signal(
                barrier_sem,
                inc=1,
                device_id=(nbr,),
                device_id_type=pl.DeviceIdType.MESH,
            )
        pl.semaphore_wait(barrier_sem, 2)

        xv = x_ref[:, :]
        scores = jnp.dot(xv, router_ref[:, :], preferred_element_type=F32)
        s_max = jnp.max(scores, axis=-1, keepdims=True)
        p_val = 1.0 / jnp.sum(jnp.exp(scores - s_max), axis=-1, keepdims=True)

        x_bf[:, :] = xv.astype(BF)
        sw_bf[:, :] = sw_ref[:, :].astype(BF)
        for j in range(E_LOCAL):
            b_bf[pl.ds(j * D_MODEL, D_MODEL), :] = ew_ref[j].astype(BF)

        pcd_all = pcd_ref[:, :]
        col_mask = (
            lax.broadcasted_iota(jnp.int32, (CAP, N_DEV), 1) == my_pos
        )
        perm_me = jnp.sum(
            jnp.where(col_mask, pcd_all, 0), axis=1, keepdims=True
        )
        gmat = (
            lax.broadcasted_iota(jnp.int32, (CAP, N_TOK), 1) == perm_me
        ).astype(BF)
        le_val = (route_ref[:, :] % E_LOCAL).astype(F32)
        meta = jnp.concatenate([p_val, le_val], axis=1).astype(BF)
        metag = jnp.dot(gmat, meta, preferred_element_type=F32)
        pg = metag[:, 0:1]
        leg = metag[:, 1:2]
        xg = jnp.dot(gmat, x_bf[:, :], preferred_element_type=F32).astype(BF)

        for j in range(E_LOCAL):
            wj = jnp.where(leg == j, pg, 0.0).astype(BF)
            a_bf[:, pl.ds(j * D_MODEL, D_MODEL)] = xg * wj
        ybuf[0, :, :] = jnp.dot(
            a_bf[:, :], b_bf[:, :], preferred_element_type=F32
        ).astype(BF)

        def scatter_add(d, slot, first):
            pdc_all = pdc_ref[:, :]
            row_mask = (
                lax.broadcasted_iota(jnp.int32, (N_DEV, CAP), 0) == d
            )
            prow = jnp.sum(
                jnp.where(row_mask, pdc_all, 0), axis=0, keepdims=True
            )
            for c in range(N_DEV):
                sl = pl.ds(c * CHUNK, CHUNK)
                smat = (
                    lax.broadcasted_iota(jnp.int32, (CHUNK, CAP), 0)
                    + c * CHUNK == prow
                ).astype(BF)
                contrib = jnp.dot(
                    smat, ybuf[slot], preferred_element_type=F32
                )
                if first:
                    shared_c = jnp.dot(
                        x_bf[sl, :], sw_bf[:, :], preferred_element_type=F32
                    )
                    out_ref[sl, :] = shared_c + contrib
                else:
                    out_ref[sl, :] = out_ref[sl, :] + contrib

        def hop(h):
            return pltpu.make_async_remote_copy(
                src_ref=ybuf.at[h],
                dst_ref=ybuf.at[h + 1],
                send_sem=send_sems.at[h],
                recv_sem=recv_sems.at[h],
                device_id=(right,),
                device_id_type=pl.DeviceIdType.MESH,
            )

        h0 = hop(0)
        h0.start()
        scatter_add(my_pos, 0, first=True)
        h0.wait_recv()

        h1 = hop(1)
        h1.start()
        scatter_add(cmod(-1), 1, first=False)
        h1.wait_recv()

        h2 = hop(2)
        h2.start()
        scatter_add(cmod(-2), 2, first=False)
        h2.wait_recv()

        scatter_add(cmod(-3), 3, first=False)

        for d in (h0, h1, h2):
            d.wait_send()

    return pl.pallas_call(
        body,
        out_shape=jax.ShapeDtypeStruct((N_TOK, D_FF), F32),
        in_specs=[pl.BlockSpec(memory_space=pltpu.VMEM)] * 7,
        out_specs=pl.BlockSpec(memory_space=pltpu.VMEM),
        scratch_shapes=[
            pltpu.VMEM((N_TOK, D_MODEL), BF),
            pltpu.VMEM((KDIM, D_FF), BF),
            pltpu.VMEM((D_MODEL, D_FF), BF),
            pltpu.VMEM((CAP, KDIM), BF),
            pltpu.VMEM((N_DEV, CAP, D_FF), BF),
            pltpu.SemaphoreType.DMA((N_DEV - 1,)),
            pltpu.SemaphoreType.DMA((N_DEV - 1,)),
        ],
        compiler_params=pltpu.CompilerParams(
            collective_id=0,
            vmem_limit_bytes=100 * 1024 * 1024,
        ),
    )(x, router_W, route_idx, expert_W, shared_W, perm_dc, perm_cd)


# device time: 79243 ns/iter; 1.2044x vs baseline; 1.2044x over previous
import jax
import jax.numpy as jnp
from jax import lax
from jax.experimental import pallas as pl
from jax.experimental.pallas import tpu as pltpu

N_DEV = 4
N_TOK = 2048
D_MODEL = 512
D_FF = 1024
E_LOCAL = 8
CHUNK = N_TOK // N_DEV
HF = D_FF // 2
KDIM = (E_LOCAL + 1) * D_MODEL
BF = jnp.bfloat16
F32 = jnp.float32


def kernel(x, router_W, route_idx, expert_W, shared_W):
    def body(
        x_ref,
        router_ref,
        route_ref,
        ew_ref,
        sw_ref,
        out_ref,
        p_ref,
        b_bf,
        a_bf,
        sndR, rsR, redR_buf, agR,
        sndL, rsL, redL_buf, agL,
        sems,
    ):
        my_pos = lax.axis_index("i")
        left = lax.rem(my_pos - 1 + N_DEV, N_DEV)
        right = lax.rem(my_pos + 1, N_DEV)

        def cmod(k):
            return lax.rem(my_pos + k + N_DEV, N_DEV)

        barrier_sem = pltpu.get_barrier_semaphore()
        for nbr in (left, right):
            pl.semaphore_signal(
                barrier_sem,
                inc=1,
                device_id=(nbr,),
                device_id_type=pl.DeviceIdType.MESH,
            )
        pl.semaphore_wait(barrier_sem, 2)

        xv = x_ref[:, :]
        scores = jnp.dot(xv, router_ref[:, :], preferred_element_type=F32)
        s_max = jnp.max(scores, axis=-1, keepdims=True)
        p_ref[:, :] = 1.0 / jnp.sum(
            jnp.exp(scores - s_max), axis=-1, keepdims=True
        )

        for j in range(E_LOCAL):
            b_bf[pl.ds(j * D_MODEL, D_MODEL), :] = ew_ref[j].astype(BF)
        b_bf[pl.ds(E_LOCAL * D_MODEL, D_MODEL), :] = (
            sw_ref[:, :] * 0.25
        ).astype(BF)

        def chunk_partial(c):
            sl = pl.ds(c * CHUNK, CHUNK)
            xc = x_ref[sl, :].astype(BF)
            routec = route_ref[sl, :]
            pc = p_ref[sl, :].astype(BF)
            for j in range(E_LOCAL):
                e = my_pos * E_LOCAL + j
                wj = jnp.where(routec == e, pc, jnp.array(0.0, BF))
                a_bf[:, pl.ds(j * D_MODEL, D_MODEL)] = xc * wj
            a_bf[:, pl.ds(E_LOCAL * D_MODEL, D_MODEL)] = xc
            acc = jnp.dot(
                a_bf[:, :], b_bf[:, :], preferred_element_type=F32
            )
            return acc[:, :HF].astype(BF), acc[:, HF:].astype(BF)

        def hop(src_ref, dst_ref, send_sem, recv_sem, to):
            return pltpu.make_async_remote_copy(
                src_ref=src_ref,
                dst_ref=dst_ref,
                send_sem=send_sem,
                recv_sem=recv_sem,
                device_id=(to,),
                device_id_type=pl.DeviceIdType.MESH,
            )

        def hopR(h):
            return hop(sndR.at[h], rsR.at[h], sems.at[0, h], sems.at[1, h],
                       right)

        def hopL(h):
            return hop(sndL.at[h], rsL.at[h], sems.at[4, h], sems.at[5, h],
                       left)

        ppL, ppR = chunk_partial(my_pos)
        sndR[0, :, :] = ppR
        sndL[0, :, :] = ppL
        r0 = hopR(0)
        l0 = hopL(0)
        r0.start()
        l0.start()

        pm1L, pm1R = chunk_partial(cmod(-1))
        pp1L, pp1R = chunk_partial(cmod(+1))

        r0.wait_recv()
        sndR[1, :, :] = (
            pm1R.astype(F32) + rsR[0][:, :].astype(F32)
        ).astype(BF)
        r1 = hopR(1)
        r1.start()
        l0.wait_recv()
        sndL[1, :, :] = (
            pp1L.astype(F32) + rsL[0][:, :].astype(F32)
        ).astype(BF)
        l1 = hopL(1)
        l1.start()

        pp2L, pp2R = chunk_partial(cmod(+2))

        r1.wait_recv()
        sndR[2, :, :] = (
            pp2R.astype(F32) + rsR[1][:, :].astype(F32)
        ).astype(BF)
        r2 = hopR(2)
        r2.start()
        l1.wait_recv()
        sndL[2, :, :] = (
            pp2L.astype(F32) + rsL[1][:, :].astype(F32)
        ).astype(BF)
        l2 = hopL(2)
        l2.start()

        r2.wait_recv()
        redR = pp1R.astype(F32) + rsR[2][:, :].astype(F32)
        redR_buf[:, :] = redR.astype(BF)
        l2.wait_recv()
        redL = pm1L.astype(F32) + rsL[2][:, :].astype(F32)
        redL_buf[:, :] = redL.astype(BF)

        ar0 = hop(redR_buf, agR.at[0], sems.at[2, 0], sems.at[3, 0], right)
        al0 = hop(redL_buf, agL.at[0], sems.at[6, 0], sems.at[7, 0], left)
        ar0.start()
        al0.start()
        out_ref[pl.ds(cmod(+1) * CHUNK, CHUNK), HF:] = redR
        out_ref[pl.ds(cmod(-1) * CHUNK, CHUNK), :HF] = redL

        ar0.wait_recv()
        ar1 = hop(agR.at[0], agR.at[1], sems.at[2, 1], sems.at[3, 1], right)
        ar1.start()
        al0.wait_recv()
        al1 = hop(agL.at[0], agL.at[1], sems.at[6, 1], sems.at[7, 1], left)
        al1.start()
        sl0 = pl.ds(my_pos * CHUNK, CHUNK)
        out_ref[sl0, HF:] = agR[0][:, :].astype(F32)
        out_ref[sl0, :HF] = agL[0][:, :].astype(F32)

        ar1.wait_recv()
        ar2 = hop(agR.at[1], agR.at[2], sems.at[2, 2], sems.at[3, 2], right)
        ar2.start()
        al1.wait_recv()
        al2 = hop(agL.at[1], agL.at[2], sems.at[6, 2], sems.at[7, 2], left)
        al2.start()
        out_ref[pl.ds(cmod(-1) * CHUNK, CHUNK), HF:] = agR[1][:, :].astype(F32)
        out_ref[pl.ds(cmod(+1) * CHUNK, CHUNK), :HF] = agL[1][:, :].astype(F32)

        ar2.wait_recv()
        al2.wait_recv()
        sl2 = pl.ds(cmod(+2) * CHUNK, CHUNK)
        out_ref[sl2, HF:] = agR[2][:, :].astype(F32)
        out_ref[sl2, :HF] = agL[2][:, :].astype(F32)

        for d in (r0, r1, r2, l0, l1, l2, ar0, ar1, ar2, al0, al1, al2):
            d.wait_send()

    return pl.pallas_call(
        body,
        out_shape=jax.ShapeDtypeStruct((N_TOK, D_FF), F32),
        in_specs=[pl.BlockSpec(memory_space=pltpu.VMEM)] * 5,
        out_specs=pl.BlockSpec(memory_space=pltpu.VMEM),
        scratch_shapes=[
            pltpu.VMEM((N_TOK, 1), F32),
            pltpu.VMEM((KDIM, D_FF), BF),
            pltpu.VMEM((CHUNK, KDIM), BF),
            pltpu.VMEM((N_DEV - 1, CHUNK, HF), BF),
            pltpu.VMEM((N_DEV - 1, CHUNK, HF), BF),
            pltpu.VMEM((CHUNK, HF), BF),
            pltpu.VMEM((N_DEV - 1, CHUNK, HF), BF),
            pltpu.VMEM((N_DEV - 1, CHUNK, HF), BF),
            pltpu.VMEM((N_DEV - 1, CHUNK, HF), BF),
            pltpu.VMEM((CHUNK, HF), BF),
            pltpu.VMEM((N_DEV - 1, CHUNK, HF), BF),
            pltpu.SemaphoreType.DMA((8, N_DEV - 1)),
        ],
        compiler_params=pltpu.CompilerParams(
            collective_id=0,
            vmem_limit_bytes=100 * 1024 * 1024,
        ),
    )(x, router_W, route_idx, expert_W, shared_W)
